# Initial kernel scaffold; baseline (speedup 1.0000x reference)
import jax
import jax.numpy as jnp
from jax.experimental import pallas as pl


def kernel(x, edge_index, edge_weight, gamma0, beta0, W1, a_src1, a_dst1, b1, gamma1, beta1, W2, a_src2, a_dst2, b2, lw1, lb1, lw2, lb2):
    raise NotImplementedError("write your pallas kernel here")


# speedup vs baseline: 4.6979x; 4.6979x over previous
import jax
import jax.numpy as jnp
from jax.experimental import pallas as pl

_N = 100000
_E = 1600000
_HEADS = 8
_HIDDEN = 16
_BE = 5000
_BN = 1000


def kernel(x, edge_index, edge_weight, gamma0, beta0, W1, a_src1, a_dst1, b1,
           gamma1, beta1, W2, a_src2, a_dst2, b2, lw1, lb1, lw2, lb2):
  x = x.astype(jnp.float32)
  xm = jnp.mean(x)
  xv = jnp.var(x)
  xn = ((x - xm) / jnp.sqrt(xv + 1e-5) * gamma0[0] + beta0[0]).reshape(_N)

  W1r = W1.reshape(_HEADS, _HIDDEN)
  c_src = jnp.sum(W1r * a_src1, axis=1)
  c_dst = jnp.sum(W1r * a_dst1, axis=1)

  src = edge_index[0]
  dst = edge_index[1]
  ew = edge_weight.astype(jnp.float32).reshape(_E, 1)

  axn = jnp.max(jnp.abs(xn))
  m1 = jnp.maximum(jnp.max(axn * (jnp.abs(c_src) + jnp.abs(c_dst))), 0.0)

  xs = xn[src].reshape(_E, 1)
  xd = xn[dst].reshape(_E, 1)

  def edge1(xs_ref, xd_ref, ew_ref, cs_ref, cd_ref, m1_ref, ex_ref, mg_ref):
    al = xs_ref[...] * cs_ref[...] + xd_ref[...] * cd_ref[...]
    al = jnp.where(al > 0, al, 0.2 * al)
    ex = jnp.exp(al - m1_ref[...])
    ex_ref[...] = ex
    mg_ref[...] = ex * (xs_ref[...] * ew_ref[...])

  eblk1 = pl.BlockSpec((_BE, 1), lambda i: (i, 0))
  cblk = pl.BlockSpec((1, _HEADS), lambda i: (0, 0))
  ex1, mg1 = pl.pallas_call(
      edge1,
      grid=(_E // _BE,),
      in_specs=[eblk1, eblk1, eblk1, cblk, cblk,
                pl.BlockSpec((1, 1), lambda i: (0, 0))],
      out_specs=[pl.BlockSpec((_BE, _HEADS), lambda i: (i, 0))] * 2,
      out_shape=[jax.ShapeDtypeStruct((_E, _HEADS), jnp.float32)] * 2,
  )(xs, xd, ew, c_src.reshape(1, _HEADS), c_dst.reshape(1, _HEADS),
    m1.reshape(1, 1))

  den1 = jax.ops.segment_sum(ex1, dst, num_segments=_N)
  num1 = jax.ops.segment_sum(mg1, dst, num_segments=_N)

  s1 = num1 / (den1 + 1e-16)
  mh = jnp.mean(s1, axis=0)
  vh = jnp.var(s1, axis=0)

  def nmid(dn_ref, nm_ref, w1_ref, g1_ref, be1_ref, w2_ref, as2_ref,
           ad2_ref, mh_ref, vh_ref, h2w_ref, asv_ref, adv_ref):
    s1b = nm_ref[...] / (dn_ref[...] + 1e-16)
    W1r_ = w1_ref[...]
    t = (s1b[:, :, None] - mh_ref[...][None, :, :1]) * W1r_[None, :, :]
    den = jnp.sqrt(vh_ref[...][None, :, :] * (W1r_ ** 2)[None, :, :] + 1e-5)
    h1n = t / den * g1_ref[...][None] + be1_ref[...][None]
    h2 = jnp.maximum(h1n, 0.0).reshape(-1, _HEADS * _HIDDEN)
    h2w = jnp.dot(h2, w2_ref[...], preferred_element_type=jnp.float32)
    h2w_ref[...] = h2w
    asv_ref[...] = jnp.sum(h2w * as2_ref[...], axis=1, keepdims=True)
    adv_ref[...] = jnp.sum(h2w * ad2_ref[...], axis=1, keepdims=True)

  hblk = pl.BlockSpec((_HEADS, _HIDDEN), lambda i: (0, 0))
  h2w, as2, ad2 = pl.pallas_call(
      nmid,
      grid=(_N // _BN,),
      in_specs=[pl.BlockSpec((_BN, _HEADS), lambda i: (i, 0)),
                pl.BlockSpec((_BN, _HEADS), lambda i: (i, 0)),
                hblk, hblk, hblk,
                pl.BlockSpec((128, 32), lambda i: (0, 0)),
                pl.BlockSpec((1, 32), lambda i: (0, 0)),
                pl.BlockSpec((1, 32), lambda i: (0, 0)),
                hblk, hblk],
      out_specs=[pl.BlockSpec((_BN, 32), lambda i: (i, 0)),
                 pl.BlockSpec((_BN, 1), lambda i: (i, 0)),
                 pl.BlockSpec((_BN, 1), lambda i: (i, 0))],
      out_shape=[jax.ShapeDtypeStruct((_N, 32), jnp.float32),
                 jax.ShapeDtypeStruct((_N, 1), jnp.float32),
                 jax.ShapeDtypeStruct((_N, 1), jnp.float32)],
  )(den1, num1, W1r, gamma1.reshape(_HEADS, _HIDDEN),
    beta1.reshape(_HEADS, _HIDDEN), W2, a_src2, a_dst2,
    jnp.broadcast_to(mh[:, None], (_HEADS, _HIDDEN)),
    jnp.broadcast_to(vh[:, None], (_HEADS, _HIDDEN)))

  as2f = as2.reshape(_N)
  ad2f = ad2.reshape(_N)
  m2 = jnp.maximum(jnp.max(as2f) + jnp.max(ad2f), 0.0)

  asg = as2f[src].reshape(_E, 1)
  adg = ad2f[dst].reshape(_E, 1)
  h2g = h2w[src]

  def edge2(as_ref, ad_ref, ew_ref, hg_ref, m2_ref, ex_ref, mm_ref):
    al = as_ref[...] + ad_ref[...]
    al = jnp.where(al > 0, al, 0.2 * al)
    ex = jnp.exp(al - m2_ref[...])
    ex_ref[...] = ex
    mm_ref[...] = hg_ref[...] * (ex * ew_ref[...])

  ex2, mm2 = pl.pallas_call(
      edge2,
      grid=(_E // _BE,),
      in_specs=[eblk1, eblk1, eblk1,
                pl.BlockSpec((_BE, 32), lambda i: (i, 0)),
                pl.BlockSpec((1, 1), lambda i: (0, 0))],
      out_specs=[pl.BlockSpec((_BE, 1), lambda i: (i, 0)),
                 pl.BlockSpec((_BE, 32), lambda i: (i, 0))],
      out_shape=[jax.ShapeDtypeStruct((_E, 1), jnp.float32),
                 jax.ShapeDtypeStruct((_E, 32), jnp.float32)],
  )(asg, adg, ew, h2g, m2.reshape(1, 1))

  den2 = jax.ops.segment_sum(ex2, dst, num_segments=_N)
  num2 = jax.ops.segment_sum(mm2, dst, num_segments=_N)

  def nout(dn_ref, nm_ref, b2_ref, lw1_ref, lb1_ref, lw2_ref, lb2_ref,
           y_ref):
    g = nm_ref[...] / (dn_ref[...] + 1e-16) + b2_ref[...]
    g = jnp.maximum(g, 0.0)
    u = jnp.maximum(
        jnp.dot(g, lw1_ref[...], preferred_element_type=jnp.float32)
        + lb1_ref[...], 0.0)
    y_ref[...] = (jnp.dot(u, lw2_ref[...], preferred_element_type=jnp.float32)
                  + lb2_ref[...])

  y = pl.pallas_call(
      nout,
      grid=(_N // _BN,),
      in_specs=[pl.BlockSpec((_BN, 1), lambda i: (i, 0)),
                pl.BlockSpec((_BN, 32), lambda i: (i, 0)),
                pl.BlockSpec((1, 32), lambda i: (0, 0)),
                pl.BlockSpec((32, 16), lambda i: (0, 0)),
                pl.BlockSpec((1, 16), lambda i: (0, 0)),
                pl.BlockSpec((16, 1), lambda i: (0, 0)),
                pl.BlockSpec((1, 1), lambda i: (0, 0))],
      out_specs=pl.BlockSpec((_BN, 1), lambda i: (i, 0)),
      out_shape=jax.ShapeDtypeStruct((_N, 1), jnp.float32),
  )(den2, num2, b2.reshape(1, 32), lw1, lb1.reshape(1, 16), lw2,
    lb2.reshape(1, 1))

  return y
